# baseline (device time: 51065 ns/iter reference)
import jax
import jax.numpy as jnp
from jax import lax
from jax.experimental import pallas as pl
from jax.experimental.pallas import tpu as pltpu


def kernel(dy, W):
    m, k = dy.shape
    d = W.shape[0]

    dyb = dy.astype(jnp.bfloat16)
    Wb = W.astype(jnp.bfloat16)

    def body(dy_ref, w_ref, out_ref, comm_ref, send_sem, recv_sem):
        my_x = lax.axis_index("x")
        my_y = lax.axis_index("y")
        my_z = lax.axis_index("z")
        partner = (my_x, my_y, 1 - my_z)

        barrier_sem = pltpu.get_barrier_semaphore()
        pl.semaphore_signal(
            barrier_sem, inc=1,
            device_id=partner, device_id_type=pl.DeviceIdType.MESH,
        )
        pl.semaphore_wait(barrier_sem, 1)

        partial = lax.dot_general(
            dy_ref[...], w_ref[...],
            (((1,), (1,)), ((), ())),
            preferred_element_type=jnp.float32,
        )
        comm_ref[0] = partial.astype(jnp.bfloat16)

        rdma = pltpu.make_async_remote_copy(
            src_ref=comm_ref.at[0],
            dst_ref=comm_ref.at[1],
            send_sem=send_sem,
            recv_sem=recv_sem,
            device_id=partner,
            device_id_type=pl.DeviceIdType.MESH,
        )
        rdma.start()
        out_ref[...] = partial
        rdma.wait()
        out_ref[...] = out_ref[...] + comm_ref[1].astype(jnp.float32)

    return pl.pallas_call(
        body,
        out_shape=jax.ShapeDtypeStruct((m, d), jnp.float32),
        in_specs=[
            pl.BlockSpec(memory_space=pltpu.VMEM),
            pl.BlockSpec(memory_space=pltpu.VMEM),
        ],
        out_specs=pl.BlockSpec(memory_space=pltpu.VMEM),
        scratch_shapes=[
            pltpu.VMEM((2, m, d), jnp.bfloat16),
            pltpu.SemaphoreType.DMA,
            pltpu.SemaphoreType.DMA,
        ],
        compiler_params=pltpu.CompilerParams(collective_id=0),
    )(dyb, Wb)


# device time: 42780 ns/iter; 1.1937x vs baseline; 1.1937x over previous
import jax
import jax.numpy as jnp
from jax import lax
from jax.experimental import pallas as pl
from jax.experimental.pallas import tpu as pltpu

M_CHUNKS = 4


def kernel(dy, W):
    m, k = dy.shape
    d = W.shape[0]
    mch = m // M_CHUNKS

    def body(dy_ref, w_ref, out_ref, wb_ref, comm_ref, send_sems, recv_sems):
        my_x = lax.axis_index("x")
        my_y = lax.axis_index("y")
        my_z = lax.axis_index("z")
        partner = (my_x, my_y, 1 - my_z)

        barrier_sem = pltpu.get_barrier_semaphore()
        pl.semaphore_signal(
            barrier_sem, inc=1,
            device_id=partner, device_id_type=pl.DeviceIdType.MESH,
        )
        pl.semaphore_wait(barrier_sem, 1)

        wb_ref[...] = w_ref[...].astype(jnp.bfloat16)

        def chunk_rdma(mc):
            sl = pl.ds(mc * mch, mch)
            return pltpu.make_async_remote_copy(
                src_ref=comm_ref.at[0, sl],
                dst_ref=comm_ref.at[1, sl],
                send_sem=send_sems.at[mc],
                recv_sem=recv_sems.at[mc],
                device_id=partner,
                device_id_type=pl.DeviceIdType.MESH,
            )

        for mc in range(M_CHUNKS):
            sl = pl.ds(mc * mch, mch)
            a = dy_ref[sl, :].astype(jnp.bfloat16)
            p = lax.dot_general(
                a, wb_ref[...],
                (((1,), (1,)), ((), ())),
                preferred_element_type=jnp.float32,
            )
            out_ref[sl, :] = p
            comm_ref[0, sl, :] = p.astype(jnp.bfloat16)
            chunk_rdma(mc).start()

        for mc in range(M_CHUNKS):
            sl = pl.ds(mc * mch, mch)
            r = chunk_rdma(mc)
            r.wait_recv()
            out_ref[sl, :] = out_ref[sl, :] + comm_ref[1, sl, :].astype(jnp.float32)
            r.wait_send()

    return pl.pallas_call(
        body,
        out_shape=jax.ShapeDtypeStruct((m, d), jnp.float32),
        in_specs=[
            pl.BlockSpec(memory_space=pltpu.VMEM),
            pl.BlockSpec(memory_space=pltpu.VMEM),
        ],
        out_specs=pl.BlockSpec(memory_space=pltpu.VMEM),
        scratch_shapes=[
            pltpu.VMEM((d, k), jnp.bfloat16),
            pltpu.VMEM((2, m, d), jnp.bfloat16),
            pltpu.SemaphoreType.DMA((M_CHUNKS,)),
            pltpu.SemaphoreType.DMA((M_CHUNKS,)),
        ],
        compiler_params=pltpu.CompilerParams(collective_id=0),
    )(dy, W)
